# initial kernel scaffold (unmeasured)
import jax
import jax.numpy as jnp
from jax import lax
from jax.experimental import pallas as pl
from jax.experimental.pallas import tpu as pltpu


def kernel(
    x,
):
    def body(*refs):
        pass

    out_shape = jax.ShapeDtypeStruct(..., jnp.float32)
    return pl.pallas_call(body, out_shape=out_shape)(...)


# baseline (device time: 411950 ns/iter reference)
import jax
import jax.numpy as jnp
from jax import lax
from jax.experimental import pallas as pl
from jax.experimental.pallas import tpu as pltpu


def kernel(x):
    xs = x[0, 0].astype(jnp.bfloat16)
    m, n = xs.shape

    def body(x_ref, out_ref, recv_y, send_sems, recv_sems):
        mx = lax.axis_index("x")
        my = lax.axis_index("y")
        x_nbr = (1 - mx, my)
        y_nbr = (mx, 1 - my)

        barrier = pltpu.get_barrier_semaphore()
        for nbr in (x_nbr, y_nbr):
            pl.semaphore_signal(
                barrier, inc=1, device_id=nbr,
                device_id_type=pl.DeviceIdType.MESH,
            )
        pl.semaphore_wait(barrier, 2)

        rdma1 = pltpu.make_async_remote_copy(
            src_ref=x_ref,
            dst_ref=out_ref,
            send_sem=send_sems.at[0],
            recv_sem=recv_sems.at[0],
            device_id=x_nbr,
            device_id_type=pl.DeviceIdType.MESH,
        )
        rdma1.start()
        rdma1.wait()
        out_ref[...] = x_ref[...] + out_ref[...]

        rdma2 = pltpu.make_async_remote_copy(
            src_ref=out_ref,
            dst_ref=recv_y,
            send_sem=send_sems.at[1],
            recv_sem=recv_sems.at[1],
            device_id=y_nbr,
            device_id_type=pl.DeviceIdType.MESH,
        )
        rdma2.start()
        rdma2.wait()
        out_ref[...] = out_ref[...] + recv_y[...]

    return pl.pallas_call(
        body,
        out_shape=jax.ShapeDtypeStruct((m, n), jnp.bfloat16),
        in_specs=[pl.BlockSpec(memory_space=pltpu.VMEM)],
        out_specs=pl.BlockSpec(memory_space=pltpu.VMEM),
        scratch_shapes=[
            pltpu.VMEM((m, n), jnp.bfloat16),
            pltpu.SemaphoreType.DMA((2,)),
            pltpu.SemaphoreType.DMA((2,)),
        ],
        compiler_params=pltpu.CompilerParams(
            collective_id=0,
            vmem_limit_bytes=100 * 1024 * 1024,
        ),
    )(xs)


# device time: 187406 ns/iter; 2.1982x vs baseline; 2.1982x over previous
import jax
import jax.numpy as jnp
from jax import lax
from jax.experimental import pallas as pl
from jax.experimental.pallas import tpu as pltpu


def kernel(x):
    xs = x[0, 0].astype(jnp.bfloat16)
    m, n = xs.shape
    q = m // 4

    def body(
        x_ref,
        out_ref,
        recv_a1,
        recv_b1,
        recv_a2,
        recv_b2,
        acc_a,
        acc_b,
        send_sems,
        recv_sems,
    ):
        mx = lax.axis_index("x")
        my = lax.axis_index("y")
        x_nbr = (1 - mx, my)
        y_nbr = (mx, 1 - my)

        a_mine = mx * q
        a_theirs = (1 - mx) * q
        b_mine = 2 * q + my * q
        b_theirs = 2 * q + (1 - my) * q

        barrier = pltpu.get_barrier_semaphore()
        for nbr in (x_nbr, y_nbr):
            pl.semaphore_signal(
                barrier, inc=1, device_id=nbr,
                device_id_type=pl.DeviceIdType.MESH,
            )
        pl.semaphore_wait(barrier, 2)

        def exch(src, dst, sem, nbr):
            return pltpu.make_async_remote_copy(
                src_ref=src, dst_ref=dst,
                send_sem=send_sems.at[sem], recv_sem=recv_sems.at[sem],
                device_id=nbr, device_id_type=pl.DeviceIdType.MESH,
            )

        p1a = exch(x_ref.at[pl.ds(a_theirs, q), :], recv_a1, 0, x_nbr)
        p1b = exch(x_ref.at[pl.ds(b_theirs, q), :], recv_b1, 1, y_nbr)
        p1a.start()
        p1b.start()
        p1a.wait()
        acc_a[...] = x_ref[pl.ds(a_mine, q), :] + recv_a1[...]
        p1b.wait()
        acc_b[...] = x_ref[pl.ds(b_mine, q), :] + recv_b1[...]

        p2a = exch(acc_a, recv_a2, 2, y_nbr)
        p2b = exch(acc_b, recv_b2, 3, x_nbr)
        p2a.start()
        p2b.start()
        p2a.wait()
        out_ref[pl.ds(a_mine, q), :] = acc_a[...] + recv_a2[...]
        p2b.wait()
        out_ref[pl.ds(b_mine, q), :] = acc_b[...] + recv_b2[...]

        p3a = exch(
            out_ref.at[pl.ds(a_mine, q), :],
            out_ref.at[pl.ds(a_mine, q), :],
            4, x_nbr,
        )
        p3b = exch(
            out_ref.at[pl.ds(b_mine, q), :],
            out_ref.at[pl.ds(b_mine, q), :],
            5, y_nbr,
        )
        p3a.start()
        p3b.start()
        p3a.wait()
        p3b.wait()

    return pl.pallas_call(
        body,
        out_shape=jax.ShapeDtypeStruct((m, n), jnp.bfloat16),
        in_specs=[pl.BlockSpec(memory_space=pltpu.VMEM)],
        out_specs=pl.BlockSpec(memory_space=pltpu.VMEM),
        scratch_shapes=[
            pltpu.VMEM((q, n), jnp.bfloat16),
            pltpu.VMEM((q, n), jnp.bfloat16),
            pltpu.VMEM((q, n), jnp.bfloat16),
            pltpu.VMEM((q, n), jnp.bfloat16),
            pltpu.VMEM((q, n), jnp.bfloat16),
            pltpu.VMEM((q, n), jnp.bfloat16),
            pltpu.SemaphoreType.DMA((6,)),
            pltpu.SemaphoreType.DMA((6,)),
        ],
        compiler_params=pltpu.CompilerParams(
            collective_id=0,
            vmem_limit_bytes=100 * 1024 * 1024,
        ),
    )(xs)


# device time: 182397 ns/iter; 2.2585x vs baseline; 1.0275x over previous
import jax
import jax.numpy as jnp
from jax import lax
from jax.experimental import pallas as pl
from jax.experimental.pallas import tpu as pltpu


def kernel(x):
    xs = x[0, 0].astype(jnp.bfloat16)
    m, n = xs.shape
    q = m // 4
    h = q // 2

    def body(
        x_ref,
        out_ref,
        recv_a1,
        recv_b1,
        recv_a2,
        recv_b2,
        acc_a,
        acc_b,
        send_sems,
        recv_sems,
    ):
        mx = lax.axis_index("x")
        my = lax.axis_index("y")
        x_nbr = (1 - mx, my)
        y_nbr = (mx, 1 - my)

        a_mine = mx * q
        a_theirs = (1 - mx) * q
        b_mine = 2 * q + my * q
        b_theirs = 2 * q + (1 - my) * q

        barrier = pltpu.get_barrier_semaphore()
        for nbr in (x_nbr, y_nbr):
            pl.semaphore_signal(
                barrier, inc=1, device_id=nbr,
                device_id_type=pl.DeviceIdType.MESH,
            )
        pl.semaphore_wait(barrier, 2)

        def exch(src, dst, sem, nbr):
            return pltpu.make_async_remote_copy(
                src_ref=src, dst_ref=dst,
                send_sem=send_sems.at[sem], recv_sem=recv_sems.at[sem],
                device_id=nbr, device_id_type=pl.DeviceIdType.MESH,
            )

        p1a = [
            exch(x_ref.at[pl.ds(a_theirs + c * h, h), :],
                 recv_a1.at[pl.ds(c * h, h), :], 0 + c, x_nbr)
            for c in range(2)
        ]
        p1b = [
            exch(x_ref.at[pl.ds(b_theirs + c * h, h), :],
                 recv_b1.at[pl.ds(c * h, h), :], 2 + c, y_nbr)
            for c in range(2)
        ]
        for r in (*p1a, *p1b):
            r.start()

        p2a = [
            exch(acc_a.at[pl.ds(c * h, h), :],
                 recv_a2.at[pl.ds(c * h, h), :], 4 + c, y_nbr)
            for c in range(2)
        ]
        p2b = [
            exch(acc_b.at[pl.ds(c * h, h), :],
                 recv_b2.at[pl.ds(c * h, h), :], 6 + c, x_nbr)
            for c in range(2)
        ]
        for c in range(2):
            s = pl.ds(c * h, h)
            p1a[c].wait()
            acc_a[s, :] = x_ref[pl.ds(a_mine + c * h, h), :] + recv_a1[s, :]
            p2a[c].start()
            p1b[c].wait()
            acc_b[s, :] = x_ref[pl.ds(b_mine + c * h, h), :] + recv_b1[s, :]
            p2b[c].start()

        p3 = []
        for c in range(2):
            s = pl.ds(c * h, h)
            p2a[c].wait()
            out_ref[pl.ds(a_mine + c * h, h), :] = acc_a[s, :] + recv_a2[s, :]
            r = exch(out_ref.at[pl.ds(a_mine + c * h, h), :],
                     out_ref.at[pl.ds(a_mine + c * h, h), :], 8 + c, x_nbr)
            r.start()
            p3.append(r)
            p2b[c].wait()
            out_ref[pl.ds(b_mine + c * h, h), :] = acc_b[s, :] + recv_b2[s, :]
            r = exch(out_ref.at[pl.ds(b_mine + c * h, h), :],
                     out_ref.at[pl.ds(b_mine + c * h, h), :], 10 + c, y_nbr)
            r.start()
            p3.append(r)
        for r in p3:
            r.wait()

    return pl.pallas_call(
        body,
        out_shape=jax.ShapeDtypeStruct((m, n), jnp.bfloat16),
        in_specs=[pl.BlockSpec(memory_space=pltpu.VMEM)],
        out_specs=pl.BlockSpec(memory_space=pltpu.VMEM),
        scratch_shapes=[
            pltpu.VMEM((q, n), jnp.bfloat16),
            pltpu.VMEM((q, n), jnp.bfloat16),
            pltpu.VMEM((q, n), jnp.bfloat16),
            pltpu.VMEM((q, n), jnp.bfloat16),
            pltpu.VMEM((q, n), jnp.bfloat16),
            pltpu.VMEM((q, n), jnp.bfloat16),
            pltpu.SemaphoreType.DMA((12,)),
            pltpu.SemaphoreType.DMA((12,)),
        ],
        compiler_params=pltpu.CompilerParams(
            collective_id=0,
            vmem_limit_bytes=100 * 1024 * 1024,
        ),
    )(xs)
